# baseline (speedup 1.0000x reference)
import functools

import jax
import jax.numpy as jnp
from jax import lax
from jax.experimental import pallas as pl
from jax.experimental.pallas import tpu as pltpu
from jax.experimental.pallas import tpu_sc as plsc

D = 300
L = 16
NVREG = 19
DP = NVREG * L
B_SEG = 256
N_ROWS = 100000
NW = 32
CHUNK = 32
NCHUNK = N_ROWS // CHUNK
NC_LO = NCHUNK // NW
NC_REM = NCHUNK - NC_LO * NW
NC_MAX = NC_LO + 1
SEG_LEN = NC_MAX * CHUNK
ACC_W = B_SEG * DP
W_LEN = DP + L


def _sc_pool_body(x_hbm, seg_hbm, w_hbm, b_hbm, out_hbm,
                  xbuf0, xbuf1, acc, segbuf, wbuf, bbuf, sem0, sem1):
    cid = lax.axis_index("c")
    sid = lax.axis_index("s")
    wid = sid * 2 + cid
    nc = NC_LO + (wid < NC_REM).astype(jnp.int32)
    gbase = NC_LO * wid + jnp.minimum(wid, NC_REM)

    pltpu.sync_copy(w_hbm, wbuf)
    pltpu.sync_copy(b_hbm, bbuf)
    pltpu.sync_copy(seg_hbm.at[pl.ds(gbase * CHUNK, SEG_LEN)], segbuf)

    zero = jnp.zeros((L,), jnp.float32)

    def zrow(i, carry):
        for u in range(8):
            acc[pl.ds((i * 8 + u) * L, L)] = zero
        return carry

    lax.fori_loop(0, ACC_W // (L * 8), zrow, 0)

    wv = [wbuf[pl.ds(k * L, L)] for k in range(NVREG - 1)]
    wtv = wbuf[pl.ds(DP, L)]
    bv = bbuf[...]
    iota = lax.iota(jnp.int32, L)
    tailmask = iota >= (L - (D - (NVREG - 1) * L))
    perms = [jnp.bitwise_xor(iota, 1 << s) for s in range(4)]

    def xcopy(g, buf, sem):
        return pltpu.async_copy(x_hbm.at[pl.ds(g * CHUNK, CHUNK), :], buf, sem)

    def xwait(buf, sem):
        pltpu.make_async_copy(x_hbm.at[pl.ds(0, CHUNK), :], buf, sem).wait()

    def process_row(buf, r, seg_scalar):
        ts = [zero, zero, zero, zero]
        xv = []
        for k in range(NVREG - 1):
            v = buf[r, pl.ds(k * L, L)]
            xv.append(v)
            ts[k & 3] = ts[k & 3] + v * wv[k]
        vt = buf[r, pl.ds(D - L, L)]
        ts[2] = ts[2] + vt * wtv
        t = (ts[0] + ts[1]) + (ts[2] + ts[3])
        for p in perms:
            t = t + t.at[p].get(mode="promise_in_bounds",
                                unique_indices=True)
        wgt = 1.0 / (1.0 + jnp.exp(-(t + bv)))
        soff = seg_scalar * DP
        for k in range(NVREG - 1):
            plsc.addupdate(acc.at[pl.ds(soff + k * L, L)], wgt * xv[k])
        tail_val = jnp.where(tailmask, wgt * vt, 0.0)
        plsc.addupdate(acc.at[pl.ds(soff + (D - L), L)], tail_val)

    def do_chunk(j, buf, sem, obuf, osem):
        @pl.when(j < nc)
        def _():
            xwait(buf, sem)

            @pl.when(j + 1 < nc)
            def _():
                xcopy(gbase + j + 1, obuf, osem)

            for h in range(CHUNK // L):
                segv = segbuf[pl.ds(j * CHUNK + h * L, L)]
                for u in range(L):
                    process_row(buf, h * L + u, segv[u])

    xcopy(gbase, xbuf0, sem0)

    def pair(jp, carry):
        do_chunk(jp * 2, xbuf0, sem0, xbuf1, sem1)
        do_chunk(jp * 2 + 1, xbuf1, sem1, xbuf0, sem0)
        return carry

    lax.fori_loop(0, (NC_MAX + 1) // 2, pair, 0)

    pltpu.sync_copy(acc, out_hbm.at[pl.ds(wid * ACC_W, ACC_W)])


_sc_pool = functools.partial(
    pl.kernel,
    out_type=jax.ShapeDtypeStruct((NW * ACC_W,), jnp.float32),
    mesh=plsc.VectorSubcoreMesh(core_axis_name="c", subcore_axis_name="s",
                                num_cores=2, num_subcores=16),
    compiler_params=pltpu.CompilerParams(use_tc_tiling_on_sc=True,
                                         needs_layout_passes=False),
    scratch_types=[
        pltpu.VMEM((CHUNK, D), jnp.float32),
        pltpu.VMEM((CHUNK, D), jnp.float32),
        pltpu.VMEM((ACC_W,), jnp.float32),
        pltpu.VMEM((SEG_LEN,), jnp.int32),
        pltpu.VMEM((W_LEN,), jnp.float32),
        pltpu.VMEM((L,), jnp.float32),
        pltpu.SemaphoreType.DMA,
        pltpu.SemaphoreType.DMA,
    ],
)(_sc_pool_body)


def _reduce_parts(p_ref, o_ref):
    o_ref[...] = jnp.sum(p_ref[...], axis=0)[:, :D]


def kernel(x, segment_ids, batch_size, W, b):
    del batch_size
    seg = jnp.pad(segment_ids.astype(jnp.int32), (0, SEG_LEN))
    wflat = W.reshape(D)
    wp = jnp.concatenate([
        wflat, jnp.zeros((DP - D + 4,), jnp.float32), wflat[D - 12:],
    ])
    bs = jnp.broadcast_to(b.reshape(1).astype(jnp.float32), (L,))
    parts = _sc_pool(x, seg, wp, bs)
    parts = parts.reshape(NW, B_SEG, DP)
    out = pl.pallas_call(
        _reduce_parts,
        out_shape=jax.ShapeDtypeStruct((B_SEG, D), jnp.float32),
    )(parts)
    return out


# speedup vs baseline: 1.5142x; 1.5142x over previous
import functools

import jax
import jax.numpy as jnp
from jax import lax
from jax.experimental import pallas as pl
from jax.experimental.pallas import tpu as pltpu
from jax.experimental.pallas import tpu_sc as plsc

D = 300
L = 16
NVREG = 19
DP = NVREG * L
B_SEG = 256
N_ROWS = 100000
NW = 32
CHUNK = 40
NCHUNK = N_ROWS // CHUNK
NC_LO = NCHUNK // NW
NC_REM = NCHUNK - NC_LO * NW
NC_MAX = NC_LO + 1
SEG_LEN = NC_MAX * CHUNK
ACC_W = B_SEG * DP
W_LEN = DP + L
RUNROLL = 4


def _sc_pool_body(x_hbm, seg_hbm, w_hbm, b_hbm, out_hbm,
                  xbuf0, xbuf1, acc, segbuf, wbuf, bbuf, sem0, sem1):
    cid = lax.axis_index("c")
    sid = lax.axis_index("s")
    wid = sid * 2 + cid
    nc = NC_LO + (wid < NC_REM).astype(jnp.int32)
    gbase = NC_LO * wid + jnp.minimum(wid, NC_REM)

    pltpu.sync_copy(w_hbm, wbuf)
    pltpu.sync_copy(b_hbm, bbuf)
    pltpu.sync_copy(seg_hbm.at[pl.ds(gbase * CHUNK, SEG_LEN)],
                    segbuf.at[pl.ds(0, SEG_LEN)])

    zero = jnp.zeros((L,), jnp.float32)

    def zrow(i, carry):
        for u in range(8):
            acc[pl.ds((i * 8 + u) * L, L)] = zero
        return carry

    lax.fori_loop(0, ACC_W // (L * 8), zrow, 0)

    wv = [wbuf[pl.ds(k * L, L)] for k in range(NVREG - 1)]
    wtv = wbuf[pl.ds(DP, L)]
    bv = bbuf[...]
    iota = lax.iota(jnp.int32, L)
    tailmask = iota >= (L - (D - (NVREG - 1) * L))
    perms = [jnp.bitwise_xor(iota, 1 << s) for s in range(4)]

    def xcopy(g, buf, sem):
        return pltpu.async_copy(x_hbm.at[pl.ds(g * CHUNK, CHUNK), :], buf, sem)

    def xwait(buf, sem):
        pltpu.make_async_copy(x_hbm.at[pl.ds(0, CHUNK), :], buf, sem).wait()

    def process_row(buf, r, seg_scalar):
        ts = [zero, zero, zero, zero]
        xv = []
        for k in range(NVREG - 1):
            v = buf[r, pl.ds(k * L, L)]
            xv.append(v)
            ts[k & 3] = ts[k & 3] + v * wv[k]
        vt = buf[r, pl.ds(D - L, L)]
        ts[2] = ts[2] + vt * wtv
        t = (ts[0] + ts[1]) + (ts[2] + ts[3])
        for p in perms:
            t = t + t.at[p].get(mode="promise_in_bounds",
                                unique_indices=True)
        wgt = 1.0 / (1.0 + jnp.exp(-(t + bv)))
        soff = seg_scalar * DP
        for k in range(NVREG - 1):
            plsc.addupdate(acc.at[pl.ds(soff + k * L, L)], wgt * xv[k])
        tail_val = jnp.where(tailmask, wgt * vt, 0.0)
        plsc.addupdate(acc.at[pl.ds(soff + (D - L), L)], tail_val)

    def do_chunk(j, buf, sem, obuf, osem):
        @pl.when(j < nc)
        def _():
            xwait(buf, sem)

            @pl.when(j + 1 < nc)
            def _():
                xcopy(gbase + j + 1, obuf, osem)

            def rows(i, carry):
                segv = segbuf[pl.ds(j * CHUNK + i * RUNROLL, L)]
                for u in range(RUNROLL):
                    process_row(buf, i * RUNROLL + u, segv[u])
                return carry

            lax.fori_loop(0, CHUNK // RUNROLL, rows, 0)

    xcopy(gbase, xbuf0, sem0)

    def pair(jp, carry):
        do_chunk(jp * 2, xbuf0, sem0, xbuf1, sem1)
        do_chunk(jp * 2 + 1, xbuf1, sem1, xbuf0, sem0)
        return carry

    lax.fori_loop(0, (NC_MAX + 1) // 2, pair, 0)

    pltpu.sync_copy(acc, out_hbm.at[pl.ds(wid * ACC_W, ACC_W)])


_sc_pool = functools.partial(
    pl.kernel,
    out_type=jax.ShapeDtypeStruct((NW * ACC_W,), jnp.float32),
    mesh=plsc.VectorSubcoreMesh(core_axis_name="c", subcore_axis_name="s",
                                num_cores=2, num_subcores=16),
    compiler_params=pltpu.CompilerParams(use_tc_tiling_on_sc=True,
                                         needs_layout_passes=False),
    scratch_types=[
        pltpu.VMEM((CHUNK, D), jnp.float32),
        pltpu.VMEM((CHUNK, D), jnp.float32),
        pltpu.VMEM((ACC_W,), jnp.float32),
        pltpu.VMEM((SEG_LEN + L,), jnp.int32),
        pltpu.VMEM((W_LEN,), jnp.float32),
        pltpu.VMEM((L,), jnp.float32),
        pltpu.SemaphoreType.DMA,
        pltpu.SemaphoreType.DMA,
    ],
)(_sc_pool_body)


def _reduce_parts(p_ref, o_ref):
    o_ref[...] = jnp.sum(p_ref[...], axis=0)[:, :D]


def kernel(x, segment_ids, batch_size, W, b):
    del batch_size
    seg = jnp.pad(segment_ids.astype(jnp.int32), (0, SEG_LEN + L))
    wflat = W.reshape(D)
    wp = jnp.concatenate([
        wflat, jnp.zeros((DP - D + 4,), jnp.float32), wflat[D - 12:],
    ])
    bs = jnp.broadcast_to(b.reshape(1).astype(jnp.float32), (L,))
    parts = _sc_pool(x, seg, wp, bs)
    parts = parts.reshape(NW, B_SEG, DP)
    out = pl.pallas_call(
        _reduce_parts,
        out_shape=jax.ShapeDtypeStruct((B_SEG, D), jnp.float32),
    )(parts)
    return out


# speedup vs baseline: 1.6404x; 1.0834x over previous
import functools

import jax
import jax.numpy as jnp
from jax import lax
from jax.experimental import pallas as pl
from jax.experimental.pallas import tpu as pltpu
from jax.experimental.pallas import tpu_sc as plsc

D = 300
L = 16
DR = 320
DH = DR // 2
B_SEG = 256
N_ROWS = 100000
NW = 32
R_BLK = 1024
N_BLK = (N_ROWS + R_BLK - 1) // R_BLK
N_PAD = N_BLK * R_BLK
CHUNK = 32
NCHUNK = N_ROWS // CHUNK
NC_LO = NCHUNK // NW
NC_REM = NCHUNK - NC_LO * NW
NC_MAX = NC_LO + 1
SEG_LEN = NC_MAX * CHUNK
ACC_W = B_SEG * DR


def _scale_block(xt_ref, wt_ref, b_ref, out_ref):
    xb = xt_ref[...]
    t = jax.lax.dot_general(wt_ref[...], xb, (((1,), (0,)), ((), ())),
                            preferred_element_type=jnp.float32)
    w = jax.nn.sigmoid(t + b_ref[0])
    s = jnp.transpose(xb * w, (1, 0))
    s = jnp.pad(s, ((0, 0), (0, DR - D)))
    be = jax.lax.bitcast_convert_type(s[:, :DH], jnp.uint32)
    bo = jax.lax.bitcast_convert_type(s[:, DH:], jnp.uint32)
    re = (be + 0x7FFF + ((be >> 16) & 1)) >> 16
    ro = (bo + 0x7FFF + ((bo >> 16) & 1)) & jnp.uint32(0xFFFF0000)
    out_ref[...] = jax.lax.bitcast_convert_type(re | ro, jnp.float32)


def _sc_pool_body(xw_hbm, seg_hbm, out_hbm, xbuf0, xbuf1, acc, segbuf,
                  sem0, sem1):
    cid = lax.axis_index("c")
    sid = lax.axis_index("s")
    wid = sid * 2 + cid
    nc = NC_LO + (wid < NC_REM).astype(jnp.int32)
    gbase = NC_LO * wid + jnp.minimum(wid, NC_REM)

    pltpu.sync_copy(seg_hbm.at[pl.ds(gbase * CHUNK, SEG_LEN)],
                    segbuf.at[pl.ds(0, SEG_LEN)])

    zero = jnp.zeros((L,), jnp.float32)

    def zrow(i, carry):
        for u in range(8):
            acc[pl.ds((i * 8 + u) * L, L)] = zero
        return carry

    lax.fori_loop(0, ACC_W // (L * 8), zrow, 0)

    himask = jnp.full((L,), 0xFFFF0000, dtype=jnp.uint32)

    def xcopy(g, buf, sem):
        return pltpu.async_copy(xw_hbm.at[pl.ds(g * CHUNK, CHUNK), :],
                                buf, sem)

    def xwait(buf, sem):
        pltpu.make_async_copy(xw_hbm.at[pl.ds(0, CHUNK), :], buf, sem).wait()

    def process_row(buf, r, seg_scalar):
        soff = seg_scalar * DR
        for k in range(DH // L):
            v = buf[r, pl.ds(k * L, L)]
            u = plsc.bitcast(v, jnp.uint32)
            lo = plsc.bitcast(u << 16, jnp.float32)
            hi = plsc.bitcast(u & himask, jnp.float32)
            plsc.addupdate(acc.at[pl.ds(soff + k * L, L)], lo)
            plsc.addupdate(acc.at[pl.ds(soff + DH + k * L, L)], hi)

    def do_chunk(j, buf, sem, obuf, osem):
        @pl.when(j < nc)
        def _():
            xwait(buf, sem)

            @pl.when(j + 1 < nc)
            def _():
                xcopy(gbase + j + 1, obuf, osem)

            def grp(h, carry):
                segv = segbuf[pl.ds(j * CHUNK + h * L, L)]
                for u in range(L):
                    process_row(buf, h * L + u, segv[u])
                return carry

            lax.fori_loop(0, CHUNK // L, grp, 0)

    xcopy(gbase, xbuf0, sem0)

    def pair(jp, carry):
        do_chunk(jp * 2, xbuf0, sem0, xbuf1, sem1)
        do_chunk(jp * 2 + 1, xbuf1, sem1, xbuf0, sem0)
        return carry

    lax.fori_loop(0, (NC_MAX + 1) // 2, pair, 0)

    pltpu.sync_copy(acc, out_hbm.at[pl.ds(wid * ACC_W, ACC_W)])


_sc_pool = functools.partial(
    pl.kernel,
    out_type=jax.ShapeDtypeStruct((NW * ACC_W,), jnp.float32),
    mesh=plsc.VectorSubcoreMesh(core_axis_name="c", subcore_axis_name="s",
                                num_cores=2, num_subcores=16),
    compiler_params=pltpu.CompilerParams(use_tc_tiling_on_sc=True,
                                         needs_layout_passes=False),
    scratch_types=[
        pltpu.VMEM((CHUNK, DH), jnp.float32),
        pltpu.VMEM((CHUNK, DH), jnp.float32),
        pltpu.VMEM((ACC_W,), jnp.float32),
        pltpu.VMEM((SEG_LEN + L,), jnp.int32),
        pltpu.SemaphoreType.DMA,
        pltpu.SemaphoreType.DMA,
    ],
)(_sc_pool_body)


def _reduce_parts(p_ref, o_ref):
    o_ref[...] = jnp.sum(p_ref[...], axis=0)[:, :D]


def kernel(x, segment_ids, batch_size, W, b):
    del batch_size
    seg = jnp.pad(segment_ids.astype(jnp.int32), (0, SEG_LEN + L))
    xw = pl.pallas_call(
        _scale_block,
        grid=(N_BLK,),
        in_specs=[
            pl.BlockSpec((D, R_BLK), lambda i: (0, i)),
            pl.BlockSpec((1, D), lambda i: (0, 0)),
            pl.BlockSpec(memory_space=pltpu.SMEM),
        ],
        out_specs=pl.BlockSpec((R_BLK, DH), lambda i: (i, 0)),
        out_shape=jax.ShapeDtypeStruct((N_PAD, DH), jnp.float32),
    )(x.T, W.reshape(1, D), b.reshape(1))
    parts = _sc_pool(xw, seg)
    parts = parts.reshape(NW, B_SEG, DR)
    out = pl.pallas_call(
        _reduce_parts,
        out_shape=jax.ShapeDtypeStruct((B_SEG, D), jnp.float32),
    )(parts)
    return out
